# Initial kernel scaffold; baseline (speedup 1.0000x reference)
import jax
import jax.numpy as jnp
from jax.experimental import pallas as pl


def kernel(x, edge_index, W_pre, b_pre, res_Wskip1, res_bskip1, res_Wskip2, res_bskip2, res_Wself1, res_Wneigh1, res_bias1, res_Wint1, res_bint1, res_Wint2, res_bint2, res_Wself2, res_Wneigh2, res_bias2, conv_Wself, conv_Wneigh, conv_bias, W_post, b_post):
    raise NotImplementedError("write your pallas kernel here")


# speedup vs baseline: 7.6993x; 7.6993x over previous
import functools

import jax
import jax.numpy as jnp
from jax import lax
from jax.experimental import pallas as pl
from jax.experimental.pallas import tpu as pltpu
from jax.experimental.pallas import tpu_sc as plsc

N = 10000
E = 320000
H = 128
NRES = 3
NCONV = 2

NW = 32
CH = 128
NCHUNK = 79
EPW = NCHUNK * CH
EPAD = NW * EPW
NPAD = 10240
RPS = NPAD // 16

BLK = 2000


def _sc_mesh():
    return plsc.VectorSubcoreMesh(core_axis_name="c", subcore_axis_name="s")


def _sc_aggregate(h, srcw, dstw):
    @functools.partial(
        pl.kernel,
        mesh=_sc_mesh(),
        out_type=jax.ShapeDtypeStruct((2, NPAD, H), jnp.float32),
        scratch_types=[
            pltpu.VMEM((NCHUNK, CH), jnp.int32),
            pltpu.VMEM((NCHUNK, CH), jnp.int32),
            pltpu.VMEM((CH, H), jnp.float32),
            pltpu.VMEM((64, H), jnp.float32),
            pltpu.VMEM_SHARED((NPAD, H), jnp.float32),
            pltpu.SemaphoreType.DMA,
        ],
    )
    def k(h_hbm, src_hbm, dst_hbm, out_hbm, sidx, didx, rows, zbuf, acc, gsem):
        c = lax.axis_index("c")
        s = lax.axis_index("s")
        wid = s * 2 + c

        @pl.loop(0, 64)
        def _zrow(r):
            @pl.loop(0, 8)
            def _zcol(q):
                zbuf[r, pl.ds(q * 16, 16)] = jnp.zeros((16,), jnp.float32)

        @pl.loop(0, RPS // 64)
        def _zacc(i):
            pltpu.sync_copy(zbuf, acc.at[pl.ds(s * RPS + i * 64, 64)])

        pltpu.sync_copy(src_hbm.at[wid], sidx)
        pltpu.sync_copy(dst_hbm.at[wid], didx)
        plsc.subcore_barrier()

        @pl.loop(0, NCHUNK)
        def _chunk(j):
            pltpu.async_copy(h_hbm.at[sidx.at[j]], rows, gsem).wait()
            pltpu.sync_copy(rows, acc.at[didx.at[j]], add=True)

        plsc.subcore_barrier()
        pltpu.sync_copy(acc.at[pl.ds(s * RPS, RPS)],
                        out_hbm.at[c, pl.ds(s * RPS, RPS)])

    return k(h, srcw, dstw)


def _sc_degree(dstw):
    @functools.partial(
        pl.kernel,
        mesh=_sc_mesh(),
        out_type=jax.ShapeDtypeStruct((2, NPAD), jnp.float32),
        scratch_types=[
            pltpu.VMEM((NCHUNK, CH), jnp.int32),
            pltpu.VMEM((CH,), jnp.float32),
            pltpu.VMEM((RPS,), jnp.float32),
            pltpu.VMEM_SHARED((NPAD,), jnp.float32),
        ],
    )
    def k(dst_hbm, out_hbm, didx, ones_v, zv, dacc):
        c = lax.axis_index("c")
        s = lax.axis_index("s")
        wid = s * 2 + c

        @pl.loop(0, CH // 16)
        def _fill(q):
            ones_v[pl.ds(q * 16, 16)] = jnp.ones((16,), jnp.float32)

        @pl.loop(0, RPS // 16)
        def _zfill(q):
            zv[pl.ds(q * 16, 16)] = jnp.zeros((16,), jnp.float32)

        pltpu.sync_copy(zv, dacc.at[pl.ds(s * RPS, RPS)])
        pltpu.sync_copy(dst_hbm.at[wid], didx)
        plsc.subcore_barrier()

        @pl.loop(0, NCHUNK)
        def _chunk(j):
            pltpu.sync_copy(ones_v, dacc.at[didx.at[j]], add=True)

        plsc.subcore_barrier()
        pltpu.sync_copy(dacc.at[pl.ds(s * RPS, RPS)],
                        out_hbm.at[c, pl.ds(s * RPS, RPS)])

    return k(dstw)


def _elu(v):
    return jnp.where(v > 0.0, v, jnp.exp(v) - 1.0)


_ROW = pl.BlockSpec((BLK, H), lambda i: (i, 0))
_ROW1 = pl.BlockSpec((BLK, 1), lambda i: (i, 0))
_P2 = pl.BlockSpec((2, BLK, H), lambda i: (0, i, 0))
_WT = pl.BlockSpec((H, H), lambda i: (0, 0))
_B1 = pl.BlockSpec((1, H), lambda i: (0, 0))
_DEG = pl.BlockSpec((BLK, 2), lambda i: (i, 0))

_ROWSHAPE = jax.ShapeDtypeStruct((N, H), jnp.float32)


def _tc_call(body, in_specs, out_specs, out_shape, args):
    out = pl.pallas_call(
        body,
        grid=(N // BLK,),
        in_specs=in_specs,
        out_specs=out_specs,
        out_shape=out_shape,
    )(*args)
    return out[0] if len(out_shape) == 1 else out


def _pre_body(x_ref, w_ref, b_ref, dp_ref, h_ref, hsl_ref, inv_ref):
    h = _elu(x_ref[...] @ w_ref[...] + b_ref[...])
    h_ref[...] = h
    hsl_ref[...] = _elu(h)
    d = dp_ref[...]
    deg = jnp.maximum(d[:, 0:1] + d[:, 1:2], 1.0)
    inv_ref[...] = 1.0 / deg


def _tc_pre(x, W_pre, b_pre, dpT):
    return _tc_call(
        _pre_body,
        [_ROW, _WT, _B1, _DEG],
        [_ROW, _ROW, _ROW1],
        [_ROWSHAPE, _ROWSHAPE, jax.ShapeDtypeStruct((N, 1), jnp.float32)],
        (x, W_pre, b_pre, dpT),
    )


def _resA_body(h_ref, p_ref, inv_ref, ws1, bs1, ws2, bs2, wse, wne, b1,
               wi1, bi1, wi2, bi2, hs_ref, h1i_ref):
    h = h_ref[...]
    hs_ref[...] = _elu(h @ ws1[...] + bs1[...]) @ ws2[...] + bs2[...]
    hn = (p_ref[0] + p_ref[1]) * inv_ref[...]
    h1 = _elu(h @ wse[...] + hn @ wne[...] + b1[...])
    h1i_ref[...] = _elu(_elu(h1 @ wi1[...] + bi1[...]) @ wi2[...] + bi2[...])


def _tc_resA(h, p, inv, ws1, bs1, ws2, bs2, wse, wne, b1, wi1, bi1, wi2, bi2):
    return _tc_call(
        _resA_body,
        [_ROW, _P2, _ROW1] + [_WT, _B1, _WT, _B1, _WT, _WT, _B1,
                              _WT, _B1, _WT, _B1],
        [_ROW, _ROW],
        [_ROWSHAPE, _ROWSHAPE],
        (h, p, inv, ws1, bs1, ws2, bs2, wse, wne, b1, wi1, bi1, wi2, bi2),
    )


def _resB_body(hs_ref, h1i_ref, p_ref, inv_ref, wse, wne, b2, out_ref):
    hn = (p_ref[0] + p_ref[1]) * inv_ref[...]
    out_ref[...] = _elu(hs_ref[...] + h1i_ref[...] @ wse[...]
                        + hn @ wne[...] + b2[...])


def _tc_resB(hs, h1i, p, inv, wse, wne, b2):
    return _tc_call(
        _resB_body,
        [_ROW, _ROW, _P2, _ROW1, _WT, _WT, _B1],
        [_ROW],
        [_ROWSHAPE],
        (hs, h1i, p, inv, wse, wne, b2),
    )


def _conv_body(h_ref, p_ref, inv_ref, ws, wn, b, out_ref):
    hn = (p_ref[0] + p_ref[1]) * inv_ref[...]
    out_ref[...] = _elu(h_ref[...] @ ws[...] + hn @ wn[...] + b[...])


def _tc_conv(h, p, inv, ws, wn, b):
    return _tc_call(
        _conv_body,
        [_ROW, _P2, _ROW1, _WT, _WT, _B1],
        [_ROW],
        [_ROWSHAPE],
        (h, p, inv, ws, wn, b),
    )


def _post_body(h_ref, hsl_ref, wa, wb, b, out_ref):
    out_ref[...] = h_ref[...] @ wa[...] + hsl_ref[...] @ wb[...] + b[...]


def _tc_post(h, hsl, wa, wb, b):
    return _tc_call(
        _post_body,
        [_ROW, _ROW, _WT, _WT, _B1],
        [_ROW],
        [_ROWSHAPE],
        (h, hsl, wa, wb, b),
    )


def kernel(x, edge_index, W_pre, b_pre, res_Wskip1, res_bskip1, res_Wskip2,
           res_bskip2, res_Wself1, res_Wneigh1, res_bias1, res_Wint1,
           res_bint1, res_Wint2, res_bint2, res_Wself2, res_Wneigh2,
           res_bias2, conv_Wself, conv_Wneigh, conv_bias, W_post, b_post):
    src = edge_index[0]
    dst = edge_index[1]
    pad = EPAD - E
    pidx = jnp.arange(pad, dtype=jnp.int32)
    srcw = jnp.concatenate([src, pidx % N]).reshape(NW, NCHUNK, CH)
    dstw = jnp.concatenate([dst, N + (pidx % 16)]).reshape(NW, NCHUNK, CH)

    dp = _sc_degree(dstw)
    dpT = dp.T[:N]
    h, hsl, inv = _tc_pre(x, W_pre, b_pre[None], dpT)

    for i in range(NRES):
        p = _sc_aggregate(h, srcw, dstw)
        hs, h1i = _tc_resA(h, p, inv,
                           res_Wskip1[i], res_bskip1[i][None],
                           res_Wskip2[i], res_bskip2[i][None],
                           res_Wself1[i], res_Wneigh1[i], res_bias1[i][None],
                           res_Wint1[i], res_bint1[i][None],
                           res_Wint2[i], res_bint2[i][None])
        p2 = _sc_aggregate(h1i, srcw, dstw)
        h = _tc_resB(hs, h1i, p2, inv,
                     res_Wself2[i], res_Wneigh2[i], res_bias2[i][None])

    for j in range(NCONV):
        p = _sc_aggregate(h, srcw, dstw)
        h = _tc_conv(h, p, inv, conv_Wself[j], conv_Wneigh[j],
                     conv_bias[j][None])

    return _tc_post(h, hsl, W_post[:H], W_post[H:], b_post[None])


# speedup vs baseline: 9.7405x; 1.2651x over previous
import functools

import jax
import jax.numpy as jnp
from jax import lax
from jax.experimental import pallas as pl
from jax.experimental.pallas import tpu as pltpu
from jax.experimental.pallas import tpu_sc as plsc

N = 10000
E = 320000
H = 128
NRES = 3
NCONV = 2

NW = 32
CH = 64
NCHUNK = 160
EPW = NCHUNK * CH
EPAD = NW * EPW
NPAD = 10240
RPS = NPAD // 16

BLK = 2000


def _sc_mesh():
    return plsc.VectorSubcoreMesh(core_axis_name="c", subcore_axis_name="s")


def _sc_aggregate(h, pkw):
    @functools.partial(
        pl.kernel,
        mesh=_sc_mesh(),
        out_type=jax.ShapeDtypeStruct((2, NPAD, H), jnp.float32),
        scratch_types=[
            pltpu.VMEM((NCHUNK, CH), jnp.int32),
            pltpu.VMEM((CH,), jnp.int32),
            pltpu.VMEM((CH,), jnp.int32),
            pltpu.VMEM((CH,), jnp.int32),
            pltpu.VMEM((CH, H), jnp.float32),
            pltpu.VMEM((CH, H), jnp.float32),
            pltpu.VMEM((16, H), jnp.float32),
            pltpu.VMEM_SHARED((NPAD, H), jnp.float32),
            pltpu.SemaphoreType.DMA,
            pltpu.SemaphoreType.DMA,
        ],
    )
    def k(h_hbm, pk_hbm, out_hbm, pidx, sidx0, sidx1, didx, rows0, rows1,
          zbuf, acc, gsem0, gsem1):
        c = lax.axis_index("c")
        s = lax.axis_index("s")
        wid = s * 2 + c

        def unpack_src(j, sb):
            @pl.loop(0, CH // 16)
            def _u(q):
                v = pidx[j, pl.ds(q * 16, 16)]
                sb[pl.ds(q * 16, 16)] = v & 16383

        def unpack_dst(j):
            @pl.loop(0, CH // 16)
            def _u(q):
                v = pidx[j, pl.ds(q * 16, 16)]
                didx[pl.ds(q * 16, 16)] = lax.shift_right_logical(v, 14)

        @pl.loop(0, 16)
        def _zrow(r):
            @pl.loop(0, 8)
            def _zcol(q):
                zbuf[r, pl.ds(q * 16, 16)] = jnp.zeros((16,), jnp.float32)

        @pl.loop(0, RPS // 16)
        def _zacc(i):
            pltpu.sync_copy(zbuf, acc.at[pl.ds(s * RPS + i * 16, 16)])

        pltpu.sync_copy(pk_hbm.at[wid], pidx)
        unpack_src(0, sidx0)
        pltpu.async_copy(h_hbm.at[sidx0], rows0, gsem0)
        plsc.subcore_barrier()

        @pl.loop(0, NCHUNK // 2)
        def _chunk(j2):
            j = j2 * 2
            unpack_src(j + 1, sidx1)
            pltpu.async_copy(h_hbm.at[sidx1], rows1, gsem1)
            pltpu.make_async_copy(h_hbm.at[sidx0], rows0, gsem0).wait()
            unpack_dst(j)
            pltpu.sync_copy(rows0, acc.at[didx], add=True)

            @pl.when(j2 + 1 < NCHUNK // 2)
            def _():
                unpack_src(j + 2, sidx0)
                pltpu.async_copy(h_hbm.at[sidx0], rows0, gsem0)

            pltpu.make_async_copy(h_hbm.at[sidx1], rows1, gsem1).wait()
            unpack_dst(j + 1)
            pltpu.sync_copy(rows1, acc.at[didx], add=True)

        plsc.subcore_barrier()
        pltpu.sync_copy(acc.at[pl.ds(s * RPS, RPS)],
                        out_hbm.at[c, pl.ds(s * RPS, RPS)])

    return k(h, pkw)


def _sc_degree(dstw):
    @functools.partial(
        pl.kernel,
        mesh=_sc_mesh(),
        out_type=jax.ShapeDtypeStruct((2, NPAD), jnp.float32),
        scratch_types=[
            pltpu.VMEM((NCHUNK, CH), jnp.int32),
            pltpu.VMEM((CH,), jnp.float32),
            pltpu.VMEM((RPS,), jnp.float32),
            pltpu.VMEM_SHARED((NPAD,), jnp.float32),
        ],
    )
    def k(dst_hbm, out_hbm, didx, ones_v, zv, dacc):
        c = lax.axis_index("c")
        s = lax.axis_index("s")
        wid = s * 2 + c

        @pl.loop(0, CH // 16)
        def _fill(q):
            ones_v[pl.ds(q * 16, 16)] = jnp.ones((16,), jnp.float32)

        @pl.loop(0, RPS // 16)
        def _zfill(q):
            zv[pl.ds(q * 16, 16)] = jnp.zeros((16,), jnp.float32)

        pltpu.sync_copy(zv, dacc.at[pl.ds(s * RPS, RPS)])
        pltpu.sync_copy(dst_hbm.at[wid], didx)
        plsc.subcore_barrier()

        @pl.loop(0, NCHUNK)
        def _chunk(j):
            pltpu.sync_copy(ones_v, dacc.at[didx.at[j]], add=True)

        plsc.subcore_barrier()
        pltpu.sync_copy(dacc.at[pl.ds(s * RPS, RPS)],
                        out_hbm.at[c, pl.ds(s * RPS, RPS)])

    return k(dstw)


def _elu(v):
    return jnp.where(v > 0.0, v, jnp.exp(v) - 1.0)


_ROW = pl.BlockSpec((BLK, H), lambda i: (i, 0))
_ROW1 = pl.BlockSpec((BLK, 1), lambda i: (i, 0))
_P2 = pl.BlockSpec((2, BLK, H), lambda i: (0, i, 0))
_WT = pl.BlockSpec((H, H), lambda i: (0, 0))
_B1 = pl.BlockSpec((1, H), lambda i: (0, 0))
_DEG = pl.BlockSpec((BLK, 2), lambda i: (i, 0))

_ROWSHAPE = jax.ShapeDtypeStruct((N, H), jnp.float32)


def _tc_call(body, in_specs, out_specs, out_shape, args):
    out = pl.pallas_call(
        body,
        grid=(N // BLK,),
        in_specs=in_specs,
        out_specs=out_specs,
        out_shape=out_shape,
    )(*args)
    return out[0] if len(out_shape) == 1 else out


def _pre_body(x_ref, w_ref, b_ref, dp_ref, h_ref, hsl_ref, inv_ref):
    h = _elu(x_ref[...] @ w_ref[...] + b_ref[...])
    h_ref[...] = h
    hsl_ref[...] = _elu(h)
    d = dp_ref[...]
    deg = jnp.maximum(d[:, 0:1] + d[:, 1:2], 1.0)
    inv_ref[...] = 1.0 / deg


def _tc_pre(x, W_pre, b_pre, dpT):
    return _tc_call(
        _pre_body,
        [_ROW, _WT, _B1, _DEG],
        [_ROW, _ROW, _ROW1],
        [_ROWSHAPE, _ROWSHAPE, jax.ShapeDtypeStruct((N, 1), jnp.float32)],
        (x, W_pre, b_pre, dpT),
    )


def _resA_body(h_ref, p_ref, inv_ref, ws1, bs1, ws2, bs2, wse, wne, b1,
               wi1, bi1, wi2, bi2, hs_ref, h1i_ref):
    h = h_ref[...]
    hs_ref[...] = _elu(h @ ws1[...] + bs1[...]) @ ws2[...] + bs2[...]
    hn = (p_ref[0] + p_ref[1]) * inv_ref[...]
    h1 = _elu(h @ wse[...] + hn @ wne[...] + b1[...])
    h1i_ref[...] = _elu(_elu(h1 @ wi1[...] + bi1[...]) @ wi2[...] + bi2[...])


def _tc_resA(h, p, inv, ws1, bs1, ws2, bs2, wse, wne, b1, wi1, bi1, wi2, bi2):
    return _tc_call(
        _resA_body,
        [_ROW, _P2, _ROW1] + [_WT, _B1, _WT, _B1, _WT, _WT, _B1,
                              _WT, _B1, _WT, _B1],
        [_ROW, _ROW],
        [_ROWSHAPE, _ROWSHAPE],
        (h, p, inv, ws1, bs1, ws2, bs2, wse, wne, b1, wi1, bi1, wi2, bi2),
    )


def _resB_body(hs_ref, h1i_ref, p_ref, inv_ref, wse, wne, b2, out_ref):
    hn = (p_ref[0] + p_ref[1]) * inv_ref[...]
    out_ref[...] = _elu(hs_ref[...] + h1i_ref[...] @ wse[...]
                        + hn @ wne[...] + b2[...])


def _tc_resB(hs, h1i, p, inv, wse, wne, b2):
    return _tc_call(
        _resB_body,
        [_ROW, _ROW, _P2, _ROW1, _WT, _WT, _B1],
        [_ROW],
        [_ROWSHAPE],
        (hs, h1i, p, inv, wse, wne, b2),
    )


def _conv_body(h_ref, p_ref, inv_ref, ws, wn, b, out_ref):
    hn = (p_ref[0] + p_ref[1]) * inv_ref[...]
    out_ref[...] = _elu(h_ref[...] @ ws[...] + hn @ wn[...] + b[...])


def _tc_conv(h, p, inv, ws, wn, b):
    return _tc_call(
        _conv_body,
        [_ROW, _P2, _ROW1, _WT, _WT, _B1],
        [_ROW],
        [_ROWSHAPE],
        (h, p, inv, ws, wn, b),
    )


def _post_body(h_ref, hsl_ref, wa, wb, b, out_ref):
    out_ref[...] = h_ref[...] @ wa[...] + hsl_ref[...] @ wb[...] + b[...]


def _tc_post(h, hsl, wa, wb, b):
    return _tc_call(
        _post_body,
        [_ROW, _ROW, _WT, _WT, _B1],
        [_ROW],
        [_ROWSHAPE],
        (h, hsl, wa, wb, b),
    )


def kernel(x, edge_index, W_pre, b_pre, res_Wskip1, res_bskip1, res_Wskip2,
           res_bskip2, res_Wself1, res_Wneigh1, res_bias1, res_Wint1,
           res_bint1, res_Wint2, res_bint2, res_Wself2, res_Wneigh2,
           res_bias2, conv_Wself, conv_Wneigh, conv_bias, W_post, b_post):
    src = edge_index[0]
    dst = edge_index[1]
    pad = EPAD - E
    pidx = jnp.arange(pad, dtype=jnp.int32)
    srcp = jnp.concatenate([src, pidx % N])
    dstp = jnp.concatenate([dst, N + (pidx % 16)])
    pkw = (dstp * 16384 + srcp).reshape(NW, NCHUNK, CH)
    dstw = dstp.reshape(NW, NCHUNK, CH)

    dp = _sc_degree(dstw)
    dpT = dp.T[:N]
    h, hsl, inv = _tc_pre(x, W_pre, b_pre[None], dpT)

    for i in range(NRES):
        p = _sc_aggregate(h, pkw)
        hs, h1i = _tc_resA(h, p, inv,
                           res_Wskip1[i], res_bskip1[i][None],
                           res_Wskip2[i], res_bskip2[i][None],
                           res_Wself1[i], res_Wneigh1[i], res_bias1[i][None],
                           res_Wint1[i], res_bint1[i][None],
                           res_Wint2[i], res_bint2[i][None])
        p2 = _sc_aggregate(h1i, pkw)
        h = _tc_resB(hs, h1i, p2, inv,
                     res_Wself2[i], res_Wneigh2[i], res_bias2[i][None])

    for j in range(NCONV):
        p = _sc_aggregate(h, pkw)
        h = _tc_conv(h, p, inv, conv_Wself[j], conv_Wneigh[j],
                     conv_bias[j][None])

    return _tc_post(h, hsl, W_post[:H], W_post[H:], b_post[None])


# speedup vs baseline: 11.6892x; 1.2001x over previous
import functools

import jax
import jax.numpy as jnp
from jax import lax
from jax.experimental import pallas as pl
from jax.experimental.pallas import tpu as pltpu
from jax.experimental.pallas import tpu_sc as plsc

N = 10000
E = 320000
H = 128
NRES = 3
NCONV = 2

NW = 32
CH = 128
NCHUNK = 80
EPW = NCHUNK * CH
EPAD = NW * EPW
NPAD = 10240
RPS = NPAD // 16

BLK = 2000


def _sc_mesh():
    return plsc.VectorSubcoreMesh(core_axis_name="c", subcore_axis_name="s")


def _sc_aggregate(h, pkw):
    @functools.partial(
        pl.kernel,
        mesh=_sc_mesh(),
        out_type=jax.ShapeDtypeStruct((2, NPAD, H), jnp.float32),
        scratch_types=[
            pltpu.VMEM((NCHUNK, CH), jnp.int32),
            pltpu.VMEM((CH,), jnp.int32),
            pltpu.VMEM((CH,), jnp.int32),
            pltpu.VMEM((CH,), jnp.int32),
            pltpu.VMEM((CH, H), jnp.float32),
            pltpu.VMEM((CH, H), jnp.float32),
            pltpu.VMEM((16, H), jnp.float32),
            pltpu.VMEM_SHARED((NPAD, H), jnp.float32),
            pltpu.SemaphoreType.DMA,
            pltpu.SemaphoreType.DMA,
        ],
    )
    def k(h_hbm, pk_hbm, out_hbm, pidx, sidx0, sidx1, didx, rows0, rows1,
          zbuf, acc, gsem0, gsem1):
        c = lax.axis_index("c")
        s = lax.axis_index("s")
        wid = s * 2 + c

        def unpack_src(j, sb):
            @pl.loop(0, CH // 16)
            def _u(q):
                v = pidx[j, pl.ds(q * 16, 16)]
                sb[pl.ds(q * 16, 16)] = v & 16383

        def unpack_dst(j):
            @pl.loop(0, CH // 16)
            def _u(q):
                v = pidx[j, pl.ds(q * 16, 16)]
                didx[pl.ds(q * 16, 16)] = lax.shift_right_logical(v, 14)

        @pl.loop(0, 16)
        def _zrow(r):
            @pl.loop(0, 8)
            def _zcol(q):
                zbuf[r, pl.ds(q * 16, 16)] = jnp.zeros((16,), jnp.float32)

        @pl.loop(0, RPS // 16)
        def _zacc(i):
            pltpu.sync_copy(zbuf, acc.at[pl.ds(s * RPS + i * 16, 16)])

        pltpu.sync_copy(pk_hbm.at[wid], pidx)
        unpack_src(0, sidx0)
        pltpu.async_copy(h_hbm.at[sidx0], rows0, gsem0)
        plsc.subcore_barrier()

        @pl.loop(0, NCHUNK // 2)
        def _chunk(j2):
            j = j2 * 2
            unpack_src(j + 1, sidx1)
            pltpu.async_copy(h_hbm.at[sidx1], rows1, gsem1)
            pltpu.make_async_copy(h_hbm.at[sidx0], rows0, gsem0).wait()
            unpack_dst(j)
            pltpu.sync_copy(rows0, acc.at[didx], add=True)

            @pl.when(j2 + 1 < NCHUNK // 2)
            def _():
                unpack_src(j + 2, sidx0)
                pltpu.async_copy(h_hbm.at[sidx0], rows0, gsem0)

            pltpu.make_async_copy(h_hbm.at[sidx1], rows1, gsem1).wait()
            unpack_dst(j + 1)
            pltpu.sync_copy(rows1, acc.at[didx], add=True)

        plsc.subcore_barrier()
        pltpu.sync_copy(acc.at[pl.ds(s * RPS, RPS)],
                        out_hbm.at[c, pl.ds(s * RPS, RPS)])

    return k(h, pkw)


def _sc_degree(dstw):
    @functools.partial(
        pl.kernel,
        mesh=_sc_mesh(),
        out_type=jax.ShapeDtypeStruct((2, NPAD), jnp.float32),
        scratch_types=[
            pltpu.VMEM((NCHUNK, CH), jnp.int32),
            pltpu.VMEM((CH,), jnp.float32),
            pltpu.VMEM((RPS,), jnp.float32),
            pltpu.VMEM_SHARED((NPAD,), jnp.float32),
        ],
    )
    def k(dst_hbm, out_hbm, didx, ones_v, zv, dacc):
        c = lax.axis_index("c")
        s = lax.axis_index("s")
        wid = s * 2 + c

        @pl.loop(0, CH // 16)
        def _fill(q):
            ones_v[pl.ds(q * 16, 16)] = jnp.ones((16,), jnp.float32)

        @pl.loop(0, RPS // 16)
        def _zfill(q):
            zv[pl.ds(q * 16, 16)] = jnp.zeros((16,), jnp.float32)

        pltpu.sync_copy(zv, dacc.at[pl.ds(s * RPS, RPS)])
        pltpu.sync_copy(dst_hbm.at[wid], didx)
        plsc.subcore_barrier()

        @pl.loop(0, NCHUNK)
        def _chunk(j):
            pltpu.sync_copy(ones_v, dacc.at[didx.at[j]], add=True)

        plsc.subcore_barrier()
        pltpu.sync_copy(dacc.at[pl.ds(s * RPS, RPS)],
                        out_hbm.at[c, pl.ds(s * RPS, RPS)])

    return k(dstw)


def _elu(v):
    return jnp.where(v > 0.0, v, jnp.exp(v) - 1.0)


_ROW = pl.BlockSpec((BLK, H), lambda i: (i, 0))
_ROW1 = pl.BlockSpec((BLK, 1), lambda i: (i, 0))
_P2 = pl.BlockSpec((2, BLK, H), lambda i: (0, i, 0))
_WT = pl.BlockSpec((H, H), lambda i: (0, 0))
_B1 = pl.BlockSpec((1, H), lambda i: (0, 0))
_DEG = pl.BlockSpec((BLK, 2), lambda i: (i, 0))

_ROWSHAPE = jax.ShapeDtypeStruct((N, H), jnp.float32)


def _tc_call(body, in_specs, out_specs, out_shape, args):
    out = pl.pallas_call(
        body,
        grid=(N // BLK,),
        in_specs=in_specs,
        out_specs=out_specs,
        out_shape=out_shape,
    )(*args)
    return out[0] if len(out_shape) == 1 else out


def _pre_body(x_ref, w_ref, b_ref, dp_ref, h_ref, hsl_ref, inv_ref):
    h = _elu(x_ref[...] @ w_ref[...] + b_ref[...])
    h_ref[...] = h
    hsl_ref[...] = _elu(h)
    d = dp_ref[...]
    deg = jnp.maximum(d[:, 0:1] + d[:, 1:2], 1.0)
    inv_ref[...] = 1.0 / deg


def _tc_pre(x, W_pre, b_pre, dpT):
    return _tc_call(
        _pre_body,
        [_ROW, _WT, _B1, _DEG],
        [_ROW, _ROW, _ROW1],
        [_ROWSHAPE, _ROWSHAPE, jax.ShapeDtypeStruct((N, 1), jnp.float32)],
        (x, W_pre, b_pre, dpT),
    )


def _resA_body(h_ref, p_ref, inv_ref, ws1, bs1, ws2, bs2, wse, wne, b1,
               wi1, bi1, wi2, bi2, hs_ref, h1i_ref):
    h = h_ref[...]
    hs_ref[...] = _elu(h @ ws1[...] + bs1[...]) @ ws2[...] + bs2[...]
    hn = (p_ref[0] + p_ref[1]) * inv_ref[...]
    h1 = _elu(h @ wse[...] + hn @ wne[...] + b1[...])
    h1i_ref[...] = _elu(_elu(h1 @ wi1[...] + bi1[...]) @ wi2[...] + bi2[...])


def _tc_resA(h, p, inv, ws1, bs1, ws2, bs2, wse, wne, b1, wi1, bi1, wi2, bi2):
    return _tc_call(
        _resA_body,
        [_ROW, _P2, _ROW1] + [_WT, _B1, _WT, _B1, _WT, _WT, _B1,
                              _WT, _B1, _WT, _B1],
        [_ROW, _ROW],
        [_ROWSHAPE, _ROWSHAPE],
        (h, p, inv, ws1, bs1, ws2, bs2, wse, wne, b1, wi1, bi1, wi2, bi2),
    )


def _resB_body(hs_ref, h1i_ref, p_ref, inv_ref, wse, wne, b2, out_ref):
    hn = (p_ref[0] + p_ref[1]) * inv_ref[...]
    out_ref[...] = _elu(hs_ref[...] + h1i_ref[...] @ wse[...]
                        + hn @ wne[...] + b2[...])


def _tc_resB(hs, h1i, p, inv, wse, wne, b2):
    return _tc_call(
        _resB_body,
        [_ROW, _ROW, _P2, _ROW1, _WT, _WT, _B1],
        [_ROW],
        [_ROWSHAPE],
        (hs, h1i, p, inv, wse, wne, b2),
    )


def _conv_body(h_ref, p_ref, inv_ref, ws, wn, b, out_ref):
    hn = (p_ref[0] + p_ref[1]) * inv_ref[...]
    out_ref[...] = _elu(h_ref[...] @ ws[...] + hn @ wn[...] + b[...])


def _tc_conv(h, p, inv, ws, wn, b):
    return _tc_call(
        _conv_body,
        [_ROW, _P2, _ROW1, _WT, _WT, _B1],
        [_ROW],
        [_ROWSHAPE],
        (h, p, inv, ws, wn, b),
    )


def _post_body(h_ref, hsl_ref, wa, wb, b, out_ref):
    out_ref[...] = h_ref[...] @ wa[...] + hsl_ref[...] @ wb[...] + b[...]


def _tc_post(h, hsl, wa, wb, b):
    return _tc_call(
        _post_body,
        [_ROW, _ROW, _WT, _WT, _B1],
        [_ROW],
        [_ROWSHAPE],
        (h, hsl, wa, wb, b),
    )


def kernel(x, edge_index, W_pre, b_pre, res_Wskip1, res_bskip1, res_Wskip2,
           res_bskip2, res_Wself1, res_Wneigh1, res_bias1, res_Wint1,
           res_bint1, res_Wint2, res_bint2, res_Wself2, res_Wneigh2,
           res_bias2, conv_Wself, conv_Wneigh, conv_bias, W_post, b_post):
    src = edge_index[0]
    dst = edge_index[1]
    pad = EPAD - E
    pidx = jnp.arange(pad, dtype=jnp.int32)
    srcp = jnp.concatenate([src, pidx % N])
    dstp = jnp.concatenate([dst, N + (pidx % 16)])
    pkw = (dstp * 16384 + srcp).reshape(NW, NCHUNK, CH)
    dstw = dstp.reshape(NW, NCHUNK, CH)

    dp = _sc_degree(dstw)
    dpT = dp.T[:N]
    h, hsl, inv = _tc_pre(x, W_pre, b_pre[None], dpT)

    for i in range(NRES):
        p = _sc_aggregate(h, pkw)
        hs, h1i = _tc_resA(h, p, inv,
                           res_Wskip1[i], res_bskip1[i][None],
                           res_Wskip2[i], res_bskip2[i][None],
                           res_Wself1[i], res_Wneigh1[i], res_bias1[i][None],
                           res_Wint1[i], res_bint1[i][None],
                           res_Wint2[i], res_bint2[i][None])
        p2 = _sc_aggregate(h1i, pkw)
        h = _tc_resB(hs, h1i, p2, inv,
                     res_Wself2[i], res_Wneigh2[i], res_bias2[i][None])

    for j in range(NCONV):
        p = _sc_aggregate(h, pkw)
        h = _tc_conv(h, p, inv, conv_Wself[j], conv_Wneigh[j],
                     conv_bias[j][None])

    return _tc_post(h, hsl, W_post[:H], W_post[H:], b_post[None])


# speedup vs baseline: 11.7569x; 1.0058x over previous
import functools

import jax
import jax.numpy as jnp
from jax import lax
from jax.experimental import pallas as pl
from jax.experimental.pallas import tpu as pltpu
from jax.experimental.pallas import tpu_sc as plsc

N = 10000
E = 320000
H = 128
NRES = 3
NCONV = 2

NW = 32
CH = 128
NCHUNK = 80
EPW = NCHUNK * CH
EPAD = NW * EPW
NPAD = 10240
RPS = NPAD // 16

BLK = 2000


def _sc_mesh():
    return plsc.VectorSubcoreMesh(core_axis_name="c", subcore_axis_name="s")


def _sc_aggregate(h, pkw):
    @functools.partial(
        pl.kernel,
        mesh=_sc_mesh(),
        out_type=jax.ShapeDtypeStruct((2, NPAD, H), jnp.float32),
        scratch_types=[
            pltpu.VMEM((NCHUNK, CH), jnp.int32),
            pltpu.VMEM((CH,), jnp.int32),
            pltpu.VMEM((CH,), jnp.int32),
            pltpu.VMEM((CH,), jnp.int32),
            pltpu.VMEM((CH, H), jnp.float32),
            pltpu.VMEM((CH, H), jnp.float32),
            pltpu.VMEM((16, H), jnp.float32),
            pltpu.VMEM_SHARED((NPAD, H), jnp.float32),
            pltpu.SemaphoreType.DMA,
            pltpu.SemaphoreType.DMA,
        ],
    )
    def k(h_hbm, pk_hbm, out_hbm, pidx, sidx0, sidx1, didx, rows0, rows1,
          zbuf, acc, gsem0, gsem1):
        c = lax.axis_index("c")
        s = lax.axis_index("s")
        wid = s * 2 + c

        def unpack_src(j, sb):
            @pl.loop(0, CH // 16)
            def _u(q):
                v = pidx[j, pl.ds(q * 16, 16)]
                sb[pl.ds(q * 16, 16)] = v & 16383

        def unpack_dst(j):
            @pl.loop(0, CH // 16)
            def _u(q):
                v = pidx[j, pl.ds(q * 16, 16)]
                didx[pl.ds(q * 16, 16)] = lax.shift_right_logical(v, 14)

        @pl.loop(0, 16)
        def _zrow(r):
            @pl.loop(0, 8)
            def _zcol(q):
                zbuf[r, pl.ds(q * 16, 16)] = jnp.zeros((16,), jnp.float32)

        @pl.loop(0, RPS // 16)
        def _zacc(i):
            pltpu.sync_copy(zbuf, acc.at[pl.ds(s * RPS + i * 16, 16)])

        pltpu.sync_copy(pk_hbm.at[wid], pidx)
        unpack_src(0, sidx0)
        pltpu.async_copy(h_hbm.at[sidx0], rows0, gsem0)
        plsc.subcore_barrier()

        @pl.loop(0, NCHUNK // 2)
        def _chunk(j2):
            j = j2 * 2
            unpack_src(j + 1, sidx1)
            pltpu.async_copy(h_hbm.at[sidx1], rows1, gsem1)
            pltpu.make_async_copy(h_hbm.at[sidx0], rows0, gsem0).wait()
            unpack_dst(j)
            pltpu.sync_copy(rows0, acc.at[didx], add=True)

            @pl.when(j2 + 1 < NCHUNK // 2)
            def _():
                unpack_src(j + 2, sidx0)
                pltpu.async_copy(h_hbm.at[sidx0], rows0, gsem0)

            pltpu.make_async_copy(h_hbm.at[sidx1], rows1, gsem1).wait()
            unpack_dst(j + 1)
            pltpu.sync_copy(rows1, acc.at[didx], add=True)

        plsc.subcore_barrier()
        pltpu.sync_copy(acc.at[pl.ds(s * RPS, RPS)],
                        out_hbm.at[c, pl.ds(s * RPS, RPS)])

    return k(h, pkw)


def _sc_degree(dstw):
    @functools.partial(
        pl.kernel,
        mesh=_sc_mesh(),
        out_type=jax.ShapeDtypeStruct((2, NPAD), jnp.float32),
        scratch_types=[
            pltpu.VMEM((NCHUNK, CH), jnp.int32),
            pltpu.VMEM((CH,), jnp.float32),
            pltpu.VMEM((RPS,), jnp.float32),
            pltpu.VMEM_SHARED((NPAD,), jnp.float32),
        ],
    )
    def k(dst_hbm, out_hbm, didx, ones_v, zv, dacc):
        c = lax.axis_index("c")
        s = lax.axis_index("s")
        wid = s * 2 + c

        @pl.loop(0, CH // 16)
        def _fill(q):
            ones_v[pl.ds(q * 16, 16)] = jnp.ones((16,), jnp.float32)

        @pl.loop(0, RPS // 16)
        def _zfill(q):
            zv[pl.ds(q * 16, 16)] = jnp.zeros((16,), jnp.float32)

        pltpu.sync_copy(zv, dacc.at[pl.ds(s * RPS, RPS)])
        pltpu.sync_copy(dst_hbm.at[wid], didx)
        plsc.subcore_barrier()

        @pl.loop(0, NCHUNK)
        def _chunk(j):
            pltpu.sync_copy(ones_v, dacc.at[didx.at[j]], add=True)

        plsc.subcore_barrier()
        pltpu.sync_copy(dacc.at[pl.ds(s * RPS, RPS)],
                        out_hbm.at[c, pl.ds(s * RPS, RPS)])

    return k(dstw)


def _elu(v):
    return jnp.where(v > 0.0, v, jnp.exp(v) - 1.0)


_ROW = pl.BlockSpec((BLK, H), lambda i: (i, 0))
_ROW1 = pl.BlockSpec((BLK, 1), lambda i: (i, 0))
_P2 = pl.BlockSpec((2, BLK, H), lambda i: (0, i, 0))
_WT = pl.BlockSpec((H, H), lambda i: (0, 0))
_B1 = pl.BlockSpec((1, H), lambda i: (0, 0))
_DEG = pl.BlockSpec((BLK, 2), lambda i: (i, 0))

_ROWSHAPE = jax.ShapeDtypeStruct((N, H), jnp.float32)


def _tc_call(body, in_specs, out_specs, out_shape, args):
    out = pl.pallas_call(
        body,
        grid=(N // BLK,),
        in_specs=in_specs,
        out_specs=out_specs,
        out_shape=out_shape,
    )(*args)
    return out[0] if len(out_shape) == 1 else out


def _pre_body(x_ref, w_ref, b_ref, dp_ref, h_ref, hsl_ref, inv_ref):
    h = _elu(x_ref[...] @ w_ref[...] + b_ref[...])
    h_ref[...] = h
    hsl_ref[...] = _elu(h)
    d = dp_ref[...]
    deg = jnp.maximum(d[:, 0:1] + d[:, 1:2], 1.0)
    inv_ref[...] = 1.0 / deg


def _tc_pre(x, W_pre, b_pre, dpT):
    return _tc_call(
        _pre_body,
        [_ROW, _WT, _B1, _DEG],
        [_ROW, _ROW, _ROW1],
        [_ROWSHAPE, _ROWSHAPE, jax.ShapeDtypeStruct((N, 1), jnp.float32)],
        (x, W_pre, b_pre, dpT),
    )


def _resHS_body(h_ref, ws1, bs1, ws2, bs2, hs_ref):
    h = h_ref[...]
    hs_ref[...] = _elu(h @ ws1[...] + bs1[...]) @ ws2[...] + bs2[...]


def _tc_resHS(h, ws1, bs1, ws2, bs2):
    return _tc_call(
        _resHS_body,
        [_ROW, _WT, _B1, _WT, _B1],
        [_ROW],
        [_ROWSHAPE],
        (h, ws1, bs1, ws2, bs2),
    )


def _resMain_body(h_ref, p_ref, inv_ref, wse, wne, b1,
                  wi1, bi1, wi2, bi2, h1i_ref):
    h = h_ref[...]
    hn = (p_ref[0] + p_ref[1]) * inv_ref[...]
    h1 = _elu(h @ wse[...] + hn @ wne[...] + b1[...])
    h1i_ref[...] = _elu(_elu(h1 @ wi1[...] + bi1[...]) @ wi2[...] + bi2[...])


def _tc_resMain(h, p, inv, wse, wne, b1, wi1, bi1, wi2, bi2):
    return _tc_call(
        _resMain_body,
        [_ROW, _P2, _ROW1, _WT, _WT, _B1, _WT, _B1, _WT, _B1],
        [_ROW],
        [_ROWSHAPE],
        (h, p, inv, wse, wne, b1, wi1, bi1, wi2, bi2),
    )


def _resB_body(hs_ref, h1i_ref, p_ref, inv_ref, wse, wne, b2, out_ref):
    hn = (p_ref[0] + p_ref[1]) * inv_ref[...]
    out_ref[...] = _elu(hs_ref[...] + h1i_ref[...] @ wse[...]
                        + hn @ wne[...] + b2[...])


def _tc_resB(hs, h1i, p, inv, wse, wne, b2):
    return _tc_call(
        _resB_body,
        [_ROW, _ROW, _P2, _ROW1, _WT, _WT, _B1],
        [_ROW],
        [_ROWSHAPE],
        (hs, h1i, p, inv, wse, wne, b2),
    )


def _conv_body(h_ref, p_ref, inv_ref, ws, wn, b, out_ref):
    hn = (p_ref[0] + p_ref[1]) * inv_ref[...]
    out_ref[...] = _elu(h_ref[...] @ ws[...] + hn @ wn[...] + b[...])


def _tc_conv(h, p, inv, ws, wn, b):
    return _tc_call(
        _conv_body,
        [_ROW, _P2, _ROW1, _WT, _WT, _B1],
        [_ROW],
        [_ROWSHAPE],
        (h, p, inv, ws, wn, b),
    )


def _post_body(h_ref, hsl_ref, wa, wb, b, out_ref):
    out_ref[...] = h_ref[...] @ wa[...] + hsl_ref[...] @ wb[...] + b[...]


def _tc_post(h, hsl, wa, wb, b):
    return _tc_call(
        _post_body,
        [_ROW, _ROW, _WT, _WT, _B1],
        [_ROW],
        [_ROWSHAPE],
        (h, hsl, wa, wb, b),
    )


def kernel(x, edge_index, W_pre, b_pre, res_Wskip1, res_bskip1, res_Wskip2,
           res_bskip2, res_Wself1, res_Wneigh1, res_bias1, res_Wint1,
           res_bint1, res_Wint2, res_bint2, res_Wself2, res_Wneigh2,
           res_bias2, conv_Wself, conv_Wneigh, conv_bias, W_post, b_post):
    src = edge_index[0]
    dst = edge_index[1]
    pad = EPAD - E
    pidx = jnp.arange(pad, dtype=jnp.int32)
    srcp = jnp.concatenate([src, pidx % N])
    dstp = jnp.concatenate([dst, N + (pidx % 16)])
    pkw = (dstp * 16384 + srcp).reshape(NW, NCHUNK, CH)
    dstw = dstp.reshape(NW, NCHUNK, CH)

    dp = _sc_degree(dstw)
    dpT = dp.T[:N]
    h, hsl, inv = _tc_pre(x, W_pre, b_pre[None], dpT)

    for i in range(NRES):
        p = _sc_aggregate(h, pkw)
        hs = _tc_resHS(h, res_Wskip1[i], res_bskip1[i][None],
                       res_Wskip2[i], res_bskip2[i][None])
        h1i = _tc_resMain(h, p, inv,
                          res_Wself1[i], res_Wneigh1[i], res_bias1[i][None],
                          res_Wint1[i], res_bint1[i][None],
                          res_Wint2[i], res_bint2[i][None])
        p2 = _sc_aggregate(h1i, pkw)
        h = _tc_resB(hs, h1i, p2, inv,
                     res_Wself2[i], res_Wneigh2[i], res_bias2[i][None])

    for j in range(NCONV):
        p = _sc_aggregate(h, pkw)
        h = _tc_conv(h, p, inv, conv_Wself[j], conv_Wneigh[j],
                     conv_bias[j][None])

    return _tc_post(h, hsl, W_post[:H], W_post[H:], b_post[None])


# speedup vs baseline: 12.1524x; 1.0336x over previous
import functools

import jax
import jax.numpy as jnp
from jax import lax
from jax.experimental import pallas as pl
from jax.experimental.pallas import tpu as pltpu
from jax.experimental.pallas import tpu_sc as plsc

N = 10000
E = 320000
H = 128
NRES = 3
NCONV = 2

NW = 32
CH = 128
NCHUNK = 80
EPW = NCHUNK * CH
EPAD = NW * EPW
NPAD = 10240
RPS = NPAD // 16

BLK = 2000


def _sc_mesh():
    return plsc.VectorSubcoreMesh(core_axis_name="c", subcore_axis_name="s")


def _sc_aggregate(h, pkw):
    @functools.partial(
        pl.kernel,
        mesh=_sc_mesh(),
        out_type=jax.ShapeDtypeStruct((2, NPAD, H), jnp.float32),
        scratch_types=[
            pltpu.VMEM((NCHUNK, CH), jnp.int32),
            pltpu.VMEM((CH,), jnp.int32),
            pltpu.VMEM((CH,), jnp.int32),
            pltpu.VMEM((CH,), jnp.int32),
            pltpu.VMEM((CH, H), jnp.float32),
            pltpu.VMEM((CH, H), jnp.float32),
            pltpu.VMEM((16, H), jnp.float32),
            pltpu.VMEM_SHARED((NPAD, H), jnp.float32),
            pltpu.SemaphoreType.DMA,
            pltpu.SemaphoreType.DMA,
            pltpu.SemaphoreType.DMA,
        ],
    )
    def k(h_hbm, pk_hbm, out_hbm, pidx, sidx0, sidx1, didx, rows0, rows1,
          zbuf, acc, gsem0, gsem1, zsem):
        c = lax.axis_index("c")
        s = lax.axis_index("s")
        wid = s * 2 + c

        def unpack_src(j, sb):
            for q in range(CH // 16):
                v = pidx[j, pl.ds(q * 16, 16)]
                sb[pl.ds(q * 16, 16)] = v & 16383

        def unpack_dst(j):
            for q in range(CH // 16):
                v = pidx[j, pl.ds(q * 16, 16)]
                didx[pl.ds(q * 16, 16)] = lax.shift_right_logical(v, 14)

        pltpu.sync_copy(pk_hbm.at[wid], pidx)
        unpack_src(0, sidx0)
        pltpu.async_copy(h_hbm.at[sidx0], rows0, gsem0)

        for r in range(16):
            for q in range(8):
                zbuf[r, pl.ds(q * 16, 16)] = jnp.zeros((16,), jnp.float32)

        for i in range(RPS // 16):
            pltpu.async_copy(zbuf, acc.at[pl.ds(s * RPS + i * 16, 16)], zsem)
        for i in range(RPS // 16):
            pltpu.make_async_copy(
                zbuf, acc.at[pl.ds(s * RPS + i * 16, 16)], zsem).wait()
        plsc.subcore_barrier()

        @pl.loop(0, NCHUNK // 2)
        def _chunk(j2):
            j = j2 * 2
            unpack_src(j + 1, sidx1)
            pltpu.async_copy(h_hbm.at[sidx1], rows1, gsem1)
            pltpu.make_async_copy(h_hbm.at[sidx0], rows0, gsem0).wait()
            unpack_dst(j)
            pltpu.sync_copy(rows0, acc.at[didx], add=True)

            @pl.when(j2 + 1 < NCHUNK // 2)
            def _():
                unpack_src(j + 2, sidx0)
                pltpu.async_copy(h_hbm.at[sidx0], rows0, gsem0)

            pltpu.make_async_copy(h_hbm.at[sidx1], rows1, gsem1).wait()
            unpack_dst(j + 1)
            pltpu.sync_copy(rows1, acc.at[didx], add=True)

        plsc.subcore_barrier()
        pltpu.sync_copy(acc.at[pl.ds(s * RPS, RPS)],
                        out_hbm.at[c, pl.ds(s * RPS, RPS)])

    return k(h, pkw)


def _sc_degree(dstw):
    @functools.partial(
        pl.kernel,
        mesh=_sc_mesh(),
        out_type=jax.ShapeDtypeStruct((2, NPAD), jnp.float32),
        scratch_types=[
            pltpu.VMEM((NCHUNK, CH), jnp.int32),
            pltpu.VMEM((CH,), jnp.float32),
            pltpu.VMEM((RPS,), jnp.float32),
            pltpu.VMEM_SHARED((NPAD,), jnp.float32),
            pltpu.SemaphoreType.DMA,
        ],
    )
    def k(dst_hbm, out_hbm, didx, ones_v, zv, dacc, ssem):
        c = lax.axis_index("c")
        s = lax.axis_index("s")
        wid = s * 2 + c

        for q in range(CH // 16):
            ones_v[pl.ds(q * 16, 16)] = jnp.ones((16,), jnp.float32)

        @pl.loop(0, RPS // 16)
        def _zfill(q):
            zv[pl.ds(q * 16, 16)] = jnp.zeros((16,), jnp.float32)

        pltpu.sync_copy(zv, dacc.at[pl.ds(s * RPS, RPS)])
        pltpu.sync_copy(dst_hbm.at[wid], didx)
        plsc.subcore_barrier()

        @pl.loop(0, NCHUNK // 8)
        def _chunk(j8):
            for u in range(8):
                pltpu.async_copy(ones_v, dacc.at[didx.at[j8 * 8 + u]], ssem,
                                 add=True)
            for u in range(8):
                pltpu.make_async_copy(
                    ones_v, dacc.at[didx.at[j8 * 8 + u]], ssem).wait()

        plsc.subcore_barrier()
        pltpu.sync_copy(dacc.at[pl.ds(s * RPS, RPS)],
                        out_hbm.at[c, pl.ds(s * RPS, RPS)])

    return k(dstw)


def _elu(v):
    return jnp.where(v > 0.0, v, jnp.exp(v) - 1.0)


_ROW = pl.BlockSpec((BLK, H), lambda i: (i, 0))
_ROW1 = pl.BlockSpec((BLK, 1), lambda i: (i, 0))
_P2 = pl.BlockSpec((2, BLK, H), lambda i: (0, i, 0))
_WT = pl.BlockSpec((H, H), lambda i: (0, 0))
_B1 = pl.BlockSpec((1, H), lambda i: (0, 0))
_DEG = pl.BlockSpec((BLK, 2), lambda i: (i, 0))

_ROWSHAPE = jax.ShapeDtypeStruct((N, H), jnp.float32)


def _tc_call(body, in_specs, out_specs, out_shape, args):
    out = pl.pallas_call(
        body,
        grid=(N // BLK,),
        in_specs=in_specs,
        out_specs=out_specs,
        out_shape=out_shape,
    )(*args)
    return out[0] if len(out_shape) == 1 else out


def _pre_body(x_ref, w_ref, b_ref, h_ref, hsl_ref):
    h = _elu(x_ref[...] @ w_ref[...] + b_ref[...])
    h_ref[...] = h
    hsl_ref[...] = _elu(h)


def _tc_pre(x, W_pre, b_pre):
    return _tc_call(
        _pre_body,
        [_ROW, _WT, _B1],
        [_ROW, _ROW],
        [_ROWSHAPE, _ROWSHAPE],
        (x, W_pre, b_pre),
    )


def _inv_body(dp_ref, inv_ref):
    d = dp_ref[...]
    deg = jnp.maximum(d[0] + d[1], 1.0)
    inv_ref[...] = (1.0 / deg)[:, None]


def _tc_inv(dp):
    return pl.pallas_call(
        _inv_body,
        grid=(NPAD // 2048,),
        in_specs=[pl.BlockSpec((2, 2048), lambda i: (0, i))],
        out_specs=pl.BlockSpec((2048, 1), lambda i: (i, 0)),
        out_shape=jax.ShapeDtypeStruct((NPAD, 1), jnp.float32),
    )(dp)


def _resHS_body(h_ref, ws1, bs1, ws2, bs2, hs_ref):
    h = h_ref[...]
    hs_ref[...] = _elu(h @ ws1[...] + bs1[...]) @ ws2[...] + bs2[...]


def _tc_resHS(h, ws1, bs1, ws2, bs2):
    return _tc_call(
        _resHS_body,
        [_ROW, _WT, _B1, _WT, _B1],
        [_ROW],
        [_ROWSHAPE],
        (h, ws1, bs1, ws2, bs2),
    )


def _resMain_body(h_ref, p_ref, inv_ref, wse, wne, b1,
                  wi1, bi1, wi2, bi2, h1i_ref):
    h = h_ref[...]
    hn = (p_ref[0] + p_ref[1]) * inv_ref[...]
    h1 = _elu(h @ wse[...] + hn @ wne[...] + b1[...])
    h1i_ref[...] = _elu(_elu(h1 @ wi1[...] + bi1[...]) @ wi2[...] + bi2[...])


def _tc_resMain(h, p, inv, wse, wne, b1, wi1, bi1, wi2, bi2):
    return _tc_call(
        _resMain_body,
        [_ROW, _P2, _ROW1, _WT, _WT, _B1, _WT, _B1, _WT, _B1],
        [_ROW],
        [_ROWSHAPE],
        (h, p, inv, wse, wne, b1, wi1, bi1, wi2, bi2),
    )


def _resB_body(hs_ref, h1i_ref, p_ref, inv_ref, wse, wne, b2, out_ref):
    hn = (p_ref[0] + p_ref[1]) * inv_ref[...]
    out_ref[...] = _elu(hs_ref[...] + h1i_ref[...] @ wse[...]
                        + hn @ wne[...] + b2[...])


def _tc_resB(hs, h1i, p, inv, wse, wne, b2):
    return _tc_call(
        _resB_body,
        [_ROW, _ROW, _P2, _ROW1, _WT, _WT, _B1],
        [_ROW],
        [_ROWSHAPE],
        (hs, h1i, p, inv, wse, wne, b2),
    )


def _conv_body(h_ref, p_ref, inv_ref, ws, wn, b, out_ref):
    hn = (p_ref[0] + p_ref[1]) * inv_ref[...]
    out_ref[...] = _elu(h_ref[...] @ ws[...] + hn @ wn[...] + b[...])


def _tc_conv(h, p, inv, ws, wn, b):
    return _tc_call(
        _conv_body,
        [_ROW, _P2, _ROW1, _WT, _WT, _B1],
        [_ROW],
        [_ROWSHAPE],
        (h, p, inv, ws, wn, b),
    )


def _post_body(h_ref, hsl_ref, wa, wb, b, out_ref):
    out_ref[...] = h_ref[...] @ wa[...] + hsl_ref[...] @ wb[...] + b[...]


def _tc_post(h, hsl, wa, wb, b):
    return _tc_call(
        _post_body,
        [_ROW, _ROW, _WT, _WT, _B1],
        [_ROW],
        [_ROWSHAPE],
        (h, hsl, wa, wb, b),
    )


def kernel(x, edge_index, W_pre, b_pre, res_Wskip1, res_bskip1, res_Wskip2,
           res_bskip2, res_Wself1, res_Wneigh1, res_bias1, res_Wint1,
           res_bint1, res_Wint2, res_bint2, res_Wself2, res_Wneigh2,
           res_bias2, conv_Wself, conv_Wneigh, conv_bias, W_post, b_post):
    src = edge_index[0]
    dst = edge_index[1]
    pad = EPAD - E
    pidx = jnp.arange(pad, dtype=jnp.int32)
    srcp = jnp.concatenate([src, pidx % N])
    dstp = jnp.concatenate([dst, N + (pidx % 16)])
    pkw = (dstp * 16384 + srcp).reshape(NW, NCHUNK, CH)
    dstw = dstp.reshape(NW, NCHUNK, CH)

    dp = _sc_degree(dstw)
    inv = _tc_inv(dp)
    h, hsl = _tc_pre(x, W_pre, b_pre[None])

    for i in range(NRES):
        p = _sc_aggregate(h, pkw)
        hs = _tc_resHS(h, res_Wskip1[i], res_bskip1[i][None],
                       res_Wskip2[i], res_bskip2[i][None])
        h1i = _tc_resMain(h, p, inv,
                          res_Wself1[i], res_Wneigh1[i], res_bias1[i][None],
                          res_Wint1[i], res_bint1[i][None],
                          res_Wint2[i], res_bint2[i][None])
        p2 = _sc_aggregate(h1i, pkw)
        h = _tc_resB(hs, h1i, p2, inv,
                     res_Wself2[i], res_Wneigh2[i], res_bias2[i][None])

    for j in range(NCONV):
        p = _sc_aggregate(h, pkw)
        h = _tc_conv(h, p, inv, conv_Wself[j], conv_Wneigh[j],
                     conv_bias[j][None])

    return _tc_post(h, hsl, W_post[:H], W_post[H:], b_post[None])
